# baseline (speedup 1.0000x reference)
import functools

import jax
import jax.numpy as jnp
from jax import lax
from jax.experimental import pallas as pl
from jax.experimental.pallas import tpu as pltpu
from jax.experimental.pallas import tpu_sc as plsc

FEAT_DIM = 512
NUM_IDS = 100000
B = 4096
L = 16


@functools.cache
def _build():
    info = plsc.get_sparse_core_info()
    nw = info.num_cores * info.num_subcores
    b_per_w = B // nw
    chunk = 64
    n_chunks = b_per_w // chunk
    vecs_per_row = FEAT_DIM // L

    mesh = plsc.VectorSubcoreMesh(core_axis_name="c", subcore_axis_name="s")

    @functools.partial(
        pl.kernel,
        mesh=mesh,
        out_type=jax.ShapeDtypeStruct((B, FEAT_DIM), jnp.float32),
        scratch_types=[
            pltpu.VMEM((chunk,), jnp.int32),
            pltpu.VMEM((chunk,), jnp.int32),
            pltpu.VMEM((chunk,), jnp.int32),
            pltpu.VMEM((chunk, FEAT_DIM), jnp.float32),
            pltpu.VMEM((chunk, FEAT_DIM), jnp.float32),
            pltpu.SemaphoreType.DMA,
        ],
    )
    def k(table_hbm, feat_hbm, mod_hbm, pid_hbm, out_hbm,
          idx_v, mod_v, pid_v, rows_v, feat_v, sem):
        wid = lax.axis_index("s") * info.num_cores + lax.axis_index("c")
        for c in range(n_chunks):
            base = wid * b_per_w + c * chunk
            pltpu.sync_copy(mod_hbm.at[pl.ds(base, chunk)], mod_v)
            pltpu.sync_copy(pid_hbm.at[pl.ds(base, chunk)], pid_v)
            for j in range(chunk // L):
                sl = pl.ds(j * L, L)
                idx_v[sl] = mod_v[sl] * NUM_IDS + pid_v[sl]
            gather = pltpu.async_copy(table_hbm.at[idx_v], rows_v, sem)
            pltpu.sync_copy(feat_hbm.at[pl.ds(base, chunk)], feat_v)
            gather.wait()

            def blend(t, carry):
                i = t // vecs_per_row
                sl2 = pl.ds((t % vecs_per_row) * L, L)
                rows_v[i, sl2] = 0.7 * feat_v[i, sl2] + 0.3 * rows_v[i, sl2]
                return carry

            lax.fori_loop(0, chunk * vecs_per_row, blend, 0)
            pltpu.sync_copy(rows_v, out_hbm.at[pl.ds(base, chunk)])

    return k


def kernel(feat, modality, pids, global_proto, local_proto):
    del local_proto
    table = global_proto.reshape(2 * NUM_IDS, FEAT_DIM)
    return _build()(table, feat, modality, pids)


# speedup vs baseline: 1.6195x; 1.6195x over previous
import functools

import jax
import jax.numpy as jnp
from jax import lax
from jax.experimental import pallas as pl
from jax.experimental.pallas import tpu as pltpu
from jax.experimental.pallas import tpu_sc as plsc

FEAT_DIM = 512
NUM_IDS = 100000
B = 4096
L = 16


@functools.cache
def _build():
    info = plsc.get_sparse_core_info()
    nw = info.num_cores * info.num_subcores
    b_per_w = B // nw
    chunk = 32
    n_chunks = b_per_w // chunk
    vecs_per_row = FEAT_DIM // L

    mesh = plsc.VectorSubcoreMesh(core_axis_name="c", subcore_axis_name="s")

    @functools.partial(
        pl.kernel,
        mesh=mesh,
        out_type=jax.ShapeDtypeStruct((B, FEAT_DIM), jnp.float32),
        scratch_types=[
            pltpu.VMEM((b_per_w,), jnp.int32),
            pltpu.VMEM((b_per_w,), jnp.int32),
            pltpu.VMEM((n_chunks, chunk), jnp.int32),
            pltpu.VMEM((chunk, FEAT_DIM), jnp.float32),
            pltpu.VMEM((chunk, FEAT_DIM), jnp.float32),
            pltpu.VMEM((chunk, FEAT_DIM), jnp.float32),
            pltpu.VMEM((chunk, FEAT_DIM), jnp.float32),
            pltpu.SemaphoreType.DMA,
            pltpu.SemaphoreType.DMA,
            pltpu.SemaphoreType.DMA,
            pltpu.SemaphoreType.DMA,
            pltpu.SemaphoreType.DMA,
            pltpu.SemaphoreType.DMA,
        ],
    )
    def k(table_hbm, feat_hbm, mod_hbm, pid_hbm, out_hbm,
          mod_v, pid_v, idx_v, rows0, rows1, feat0, feat1,
          gsem0, gsem1, fsem0, fsem1, osem0, osem1):
        rows = (rows0, rows1)
        feats = (feat0, feat1)
        gsems = (gsem0, gsem1)
        fsems = (fsem0, fsem1)
        osems = (osem0, osem1)

        wid = lax.axis_index("s") * info.num_cores + lax.axis_index("c")
        base = wid * b_per_w
        pltpu.sync_copy(mod_hbm.at[pl.ds(base, b_per_w)], mod_v)
        pltpu.sync_copy(pid_hbm.at[pl.ds(base, b_per_w)], pid_v)
        for j in range(b_per_w // L):
            sl = pl.ds(j * L, L)
            idx_v[j // (chunk // L), pl.ds((j % (chunk // L)) * L, L)] = (
                mod_v[sl] * NUM_IDS + pid_v[sl])

        def start(c):
            b = c % 2
            g = pltpu.async_copy(table_hbm.at[idx_v.at[c]], rows[b], gsems[b])
            f = pltpu.async_copy(
                feat_hbm.at[pl.ds(base + c * chunk, chunk)], feats[b], fsems[b])
            return g, f

        inflight = [start(0), start(1)]
        out_cp = [None] * n_chunks
        for c in range(n_chunks):
            b = c % 2
            g, f = inflight[c % 2]
            g.wait()
            f.wait()
            rb, fb = rows[b], feats[b]

            def blend_row(i, carry):
                for v in range(vecs_per_row):
                    sl = pl.ds(v * L, L)
                    rb[i, sl] = 0.7 * fb[i, sl] + 0.3 * rb[i, sl]
                return carry

            lax.fori_loop(0, chunk, blend_row, 0)
            out_cp[c] = pltpu.async_copy(
                rb, out_hbm.at[pl.ds(base + c * chunk, chunk)], osems[b])
            if c + 2 < n_chunks:
                out_cp[c].wait()
                inflight[c % 2] = start(c + 2)
        out_cp[n_chunks - 2].wait()
        out_cp[n_chunks - 1].wait()

    return k


def kernel(feat, modality, pids, global_proto, local_proto):
    del local_proto
    table = global_proto.reshape(2 * NUM_IDS, FEAT_DIM)
    return _build()(table, feat, modality, pids)
